# baseline (speedup 1.0000x reference)
import functools

import jax
import jax.numpy as jnp
from jax import lax
from jax.experimental import pallas as pl
from jax.experimental.pallas import tpu as pltpu
from jax.experimental.pallas import tpu_sc as plsc

NUM_CORES = 2
NUM_SUBCORES = 16
NUM_WORKERS = NUM_CORES * NUM_SUBCORES


def _scatter_sum_sc(ei_flat, edge_attr, n_nodes):
    E, H = edge_attr.shape
    N = n_nodes
    assert E % NUM_WORKERS == 0
    epw = E // NUM_WORKERS
    CH = 80
    assert epw % CH == 0
    n_chunks = epw // CH
    NBUF = 4
    PDIST = 3
    rows_per_tile = -(-N // (NUM_SUBCORES * 8)) * 8
    n_pad = rows_per_tile * NUM_SUBCORES
    cnt_per_tile = -(-N // (NUM_SUBCORES * 16)) * 16
    cnt_pad = cnt_per_tile * NUM_SUBCORES

    mesh = plsc.VectorSubcoreMesh(core_axis_name="c", subcore_axis_name="s")

    zero_rows = jnp.zeros((rows_per_tile, H), jnp.float32)
    zero_cnt = jnp.zeros((cnt_per_tile,), jnp.float32)

    @functools.partial(
        pl.kernel,
        out_type=(
            jax.ShapeDtypeStruct((NUM_CORES, n_pad, H), jnp.float32),
            jax.ShapeDtypeStruct((NUM_CORES, cnt_pad), jnp.float32),
        ),
        mesh=mesh,
        scratch_types=[
            pltpu.VMEM((NBUF, CH, H), jnp.float32),
            pltpu.VMEM((NBUF, CH), jnp.int32),
            pltpu.VMEM((CH,), jnp.float32),
            pltpu.VMEM_SHARED((n_pad, H), jnp.float32),
            pltpu.VMEM_SHARED((cnt_pad,), jnp.float32),
            pltpu.SemaphoreType.DMA((NBUF,)),
            pltpu.SemaphoreType.DMA((NBUF,)),
            pltpu.SemaphoreType.DMA((NBUF,)),
            pltpu.SemaphoreType.DMA((NBUF,)),
        ],
    )
    def sc_kernel(col_hbm, ea_hbm, zrows_hbm, zcnt_hbm,
                  psum_hbm, cnt_hbm,
                  ebuf, ibuf, ones, acc, cnt_sh, esem, isem, ssem, csem):
        c = lax.axis_index("c")
        s = lax.axis_index("s")
        w = c * NUM_SUBCORES + s
        base = w * epw
        cbase = E + base

        def fill_ones(i, _):
            ones[pl.ds(i * 16, 16)] = jnp.ones((16,), jnp.float32)
            return 0
        lax.fori_loop(0, CH // 16, fill_ones, 0)

        def start_loads(k, slot):
            pltpu.async_copy(col_hbm.at[pl.ds(cbase + k * CH, CH)],
                             ibuf.at[slot], isem.at[slot])
            pltpu.async_copy(ea_hbm.at[pl.ds(base + k * CH, CH)],
                             ebuf.at[slot], esem.at[slot])

        def wait_loads(k, slot):
            pltpu.make_async_copy(col_hbm.at[pl.ds(cbase + k * CH, CH)],
                                  ibuf.at[slot], isem.at[slot]).wait()
            pltpu.make_async_copy(ea_hbm.at[pl.ds(base + k * CH, CH)],
                                  ebuf.at[slot], esem.at[slot]).wait()

        def drain_scatters(k, slot):
            pltpu.make_async_copy(ebuf.at[slot], acc.at[ibuf.at[slot]],
                                  ssem.at[slot]).wait()
            pltpu.make_async_copy(ones, cnt_sh.at[ibuf.at[slot]],
                                  csem.at[slot]).wait()

        def issue_scatters(k, slot):
            pltpu.async_copy(ebuf.at[slot], acc.at[ibuf.at[slot]],
                             ssem.at[slot], add=True)
            pltpu.async_copy(ones, cnt_sh.at[ibuf.at[slot]],
                             csem.at[slot], add=True)

        for kp in range(PDIST):
            start_loads(kp, kp)
        pltpu.sync_copy(zrows_hbm, acc.at[pl.ds(s * rows_per_tile, rows_per_tile)])
        pltpu.sync_copy(zcnt_hbm, cnt_sh.at[pl.ds(s * cnt_per_tile, cnt_per_tile)])
        plsc.subcore_barrier()

        wait_loads(0, 0)
        issue_scatters(0, 0)
        start_loads(PDIST, PDIST % NBUF)

        def chunk_body(k, _):
            b = lax.rem(k, NBUF)
            wait_loads(k, b)
            issue_scatters(k, b)
            kp = k + PDIST
            bp = lax.rem(kp, NBUF)
            drain_scatters(kp - NBUF, bp)
            start_loads(kp, bp)
            return 0
        lax.fori_loop(1, n_chunks - PDIST, chunk_body, 0)

        for k in range(n_chunks - PDIST, n_chunks):
            b = k % NBUF
            wait_loads(k, b)
            issue_scatters(k, b)

        for t in range(NBUF):
            kd = n_chunks - NBUF + t
            drain_scatters(kd, kd % NBUF)

        plsc.subcore_barrier()

        pltpu.sync_copy(acc.at[pl.ds(s * rows_per_tile, rows_per_tile)],
                        psum_hbm.at[c, pl.ds(s * rows_per_tile, rows_per_tile)])
        pltpu.sync_copy(cnt_sh.at[pl.ds(s * cnt_per_tile, cnt_per_tile)],
                        cnt_hbm.at[c, pl.ds(s * cnt_per_tile, cnt_per_tile)])

    return sc_kernel(ei_flat, edge_attr, zero_rows, zero_cnt)


def _mlp_tc(x, psum, cnt, W1T, b1, gamma, beta, prelu_w, W2T, b2):
    N, H = x.shape
    H2 = 2 * H
    R = 2000
    assert N % R == 0
    grid = (N // R,)

    def body(x_ref, p_ref, c_ref, w1_ref, b1_ref, g_ref, be_ref, pw_ref,
             w2_ref, b2_ref, y_ref):
        psum_blk = p_ref[0] + p_ref[1]
        cc = c_ref[..., 0] + c_ref[..., 1]
        mean = psum_blk / jnp.clip(cc, 1.0, None)[:, None]
        out = jnp.concatenate([x_ref[...], mean], axis=1)
        h = jnp.dot(out.astype(jnp.bfloat16), w1_ref[...],
                    preferred_element_type=jnp.float32) + b1_ref[...]
        mu = jnp.mean(h, axis=1, keepdims=True)
        var = jnp.mean((h - mu) ** 2, axis=1, keepdims=True)
        ln = g_ref[...] * (h - mu) / jnp.sqrt(var + 1e-5) + be_ref[...]
        pw = pw_ref[0, 0]
        pr = jnp.where(ln >= 0, ln, pw * ln)
        y_ref[...] = jnp.dot(pr.astype(jnp.bfloat16), w2_ref[...],
                             preferred_element_type=jnp.float32) + b2_ref[...]

    return pl.pallas_call(
        body,
        grid=grid,
        in_specs=[
            pl.BlockSpec((R, H), lambda i: (i, 0)),
            pl.BlockSpec((NUM_CORES, R, H), lambda i: (0, i, 0)),
            pl.BlockSpec((R, NUM_CORES), lambda i: (i, 0)),
            pl.BlockSpec((H2, H2), lambda i: (0, 0)),
            pl.BlockSpec((1, H2), lambda i: (0, 0)),
            pl.BlockSpec((1, H2), lambda i: (0, 0)),
            pl.BlockSpec((1, H2), lambda i: (0, 0)),
            pl.BlockSpec(memory_space=pltpu.SMEM),
            pl.BlockSpec((H2, H), lambda i: (0, 0)),
            pl.BlockSpec((1, H), lambda i: (0, 0)),
        ],
        out_specs=pl.BlockSpec((R, H), lambda i: (i, 0)),
        out_shape=jax.ShapeDtypeStruct((N, H), jnp.float32),
    )(x, psum, cnt, W1T, b1, gamma, beta, prelu_w, W2T, b2)


def kernel(x, edge_index, edge_attr, W1, b1, gamma, beta, prelu_w, W2, b2):
    N, H = x.shape
    psum, cnt = _scatter_sum_sc(edge_index.reshape(-1), edge_attr, N)
    cnt = cnt[:, :N].T
    return _mlp_tc(
        x, psum, cnt,
        W1.T.astype(jnp.bfloat16), b1.reshape(1, -1), gamma.reshape(1, -1),
        beta.reshape(1, -1), prelu_w.reshape(1, 1),
        W2.T.astype(jnp.bfloat16), b2.reshape(1, -1),
    )


# speedup vs baseline: 1.0182x; 1.0182x over previous
import functools

import jax
import jax.numpy as jnp
from jax import lax
from jax.experimental import pallas as pl
from jax.experimental.pallas import tpu as pltpu
from jax.experimental.pallas import tpu_sc as plsc

NUM_CORES = 2
NUM_SUBCORES = 16
NUM_WORKERS = NUM_CORES * NUM_SUBCORES


def _scatter_sum_sc(ei_flat, edge_attr, n_nodes):
    E, H = edge_attr.shape
    N = n_nodes
    assert E % NUM_WORKERS == 0
    epw = E // NUM_WORKERS
    CH = 80
    assert epw % CH == 0
    n_chunks = epw // CH
    NBUF = 4
    PDIST = 3
    rows_per_tile = -(-N // (NUM_SUBCORES * 8)) * 8
    n_pad = rows_per_tile * NUM_SUBCORES
    cnt_per_tile = -(-N // (NUM_SUBCORES * 16)) * 16
    cnt_pad = cnt_per_tile * NUM_SUBCORES

    mesh = plsc.VectorSubcoreMesh(core_axis_name="c", subcore_axis_name="s")

    zero_rows = jnp.zeros((rows_per_tile, H), jnp.float32)
    zero_cnt = jnp.zeros((cnt_per_tile,), jnp.float32)

    @functools.partial(
        pl.kernel,
        out_type=(
            jax.ShapeDtypeStruct((NUM_CORES, n_pad, H), jnp.float32),
            jax.ShapeDtypeStruct((NUM_CORES, cnt_pad), jnp.float32),
        ),
        mesh=mesh,
        scratch_types=[
            pltpu.VMEM((NBUF, CH, H), jnp.float32),
            pltpu.VMEM((NBUF, CH), jnp.int32),
            pltpu.VMEM((CH,), jnp.float32),
            pltpu.VMEM_SHARED((n_pad, H), jnp.float32),
            pltpu.VMEM_SHARED((cnt_pad,), jnp.float32),
            pltpu.SemaphoreType.DMA((NBUF,)),
            pltpu.SemaphoreType.DMA((NBUF,)),
            pltpu.SemaphoreType.DMA((NBUF,)),
            pltpu.SemaphoreType.DMA((NBUF,)),
        ],
    )
    def sc_kernel(col_hbm, ea_hbm, zrows_hbm, zcnt_hbm,
                  psum_hbm, cnt_hbm,
                  ebuf, ibuf, ones, acc, cnt_sh, esem, isem, ssem, csem):
        c = lax.axis_index("c")
        s = lax.axis_index("s")
        w = c * NUM_SUBCORES + s
        base = w * epw
        cbase = E + base

        def fill_ones(i, _):
            ones[pl.ds(i * 16, 16)] = jnp.ones((16,), jnp.float32)
            return 0
        lax.fori_loop(0, CH // 16, fill_ones, 0)

        def start_loads(k, slot):
            pltpu.async_copy(col_hbm.at[pl.ds(cbase + k * CH, CH)],
                             ibuf.at[slot], isem.at[slot])
            pltpu.async_copy(ea_hbm.at[pl.ds(base + k * CH, CH)],
                             ebuf.at[slot], esem.at[slot])

        def wait_loads(k, slot):
            pltpu.make_async_copy(col_hbm.at[pl.ds(cbase + k * CH, CH)],
                                  ibuf.at[slot], isem.at[slot]).wait()
            pltpu.make_async_copy(ea_hbm.at[pl.ds(base + k * CH, CH)],
                                  ebuf.at[slot], esem.at[slot]).wait()

        def drain_scatters(k, slot):
            pltpu.make_async_copy(ebuf.at[slot], acc.at[ibuf.at[slot]],
                                  ssem.at[slot]).wait()
            pltpu.make_async_copy(ones, cnt_sh.at[ibuf.at[slot]],
                                  csem.at[slot]).wait()

        def issue_scatters(k, slot):
            pltpu.async_copy(ebuf.at[slot], acc.at[ibuf.at[slot]],
                             ssem.at[slot], add=True)
            pltpu.async_copy(ones, cnt_sh.at[ibuf.at[slot]],
                             csem.at[slot], add=True)

        for kp in range(PDIST):
            start_loads(kp, kp)
        pltpu.sync_copy(zrows_hbm, acc.at[pl.ds(s * rows_per_tile, rows_per_tile)])
        pltpu.sync_copy(zcnt_hbm, cnt_sh.at[pl.ds(s * cnt_per_tile, cnt_per_tile)])
        plsc.subcore_barrier()

        wait_loads(0, 0)
        issue_scatters(0, 0)
        start_loads(PDIST, PDIST % NBUF)

        def chunk_body(k, _):
            b = lax.rem(k, NBUF)
            wait_loads(k, b)
            issue_scatters(k, b)
            kp = k + PDIST
            bp = lax.rem(kp, NBUF)
            drain_scatters(kp - NBUF, bp)
            start_loads(kp, bp)
            return 0
        lax.fori_loop(1, n_chunks - PDIST, chunk_body, 0)

        for k in range(n_chunks - PDIST, n_chunks):
            b = k % NBUF
            wait_loads(k, b)
            issue_scatters(k, b)

        for t in range(NBUF):
            kd = n_chunks - NBUF + t
            drain_scatters(kd, kd % NBUF)

        plsc.subcore_barrier()

        pltpu.sync_copy(acc.at[pl.ds(s * rows_per_tile, rows_per_tile)],
                        psum_hbm.at[c, pl.ds(s * rows_per_tile, rows_per_tile)])
        pltpu.sync_copy(cnt_sh.at[pl.ds(s * cnt_per_tile, cnt_per_tile)],
                        cnt_hbm.at[c, pl.ds(s * cnt_per_tile, cnt_per_tile)])

    return sc_kernel(ei_flat, edge_attr, zero_rows, zero_cnt)


def _mlp_tc(x, psum, cnt, W1T, b1, gamma, beta, prelu_w, W2T, b2):
    N, H = x.shape
    H2 = 2 * H
    R = 1024
    grid = (pl.cdiv(N, R),)

    def body(x_ref, p_ref, c_ref, w1_ref, b1_ref, g_ref, be_ref, pw_ref,
             w2_ref, b2_ref, y_ref):
        psum_blk = p_ref[0] + p_ref[1]
        cc = c_ref[0] + c_ref[1]
        mean = psum_blk / jnp.clip(cc, 1.0, None)[:, None]
        out = jnp.concatenate([x_ref[...], mean], axis=1)
        h = jnp.dot(out, w1_ref[...], preferred_element_type=jnp.float32) + b1_ref[...]
        mu = jnp.mean(h, axis=1, keepdims=True)
        var = jnp.mean((h - mu) ** 2, axis=1, keepdims=True)
        ln = g_ref[...] * (h - mu) / jnp.sqrt(var + 1e-5) + be_ref[...]
        pw = pw_ref[0, 0]
        pr = jnp.where(ln >= 0, ln, pw * ln)
        y_ref[...] = jnp.dot(pr, w2_ref[...], preferred_element_type=jnp.float32) + b2_ref[...]

    return pl.pallas_call(
        body,
        grid=grid,
        in_specs=[
            pl.BlockSpec((R, H), lambda i: (i, 0)),
            pl.BlockSpec((NUM_CORES, R, H), lambda i: (0, i, 0)),
            pl.BlockSpec((NUM_CORES, R), lambda i: (0, i)),
            pl.BlockSpec((H2, H2), lambda i: (0, 0)),
            pl.BlockSpec((1, H2), lambda i: (0, 0)),
            pl.BlockSpec((1, H2), lambda i: (0, 0)),
            pl.BlockSpec((1, H2), lambda i: (0, 0)),
            pl.BlockSpec(memory_space=pltpu.SMEM),
            pl.BlockSpec((H2, H), lambda i: (0, 0)),
            pl.BlockSpec((1, H), lambda i: (0, 0)),
        ],
        out_specs=pl.BlockSpec((R, H), lambda i: (i, 0)),
        out_shape=jax.ShapeDtypeStruct((N, H), jnp.float32),
    )(x, psum, cnt, W1T, b1, gamma, beta, prelu_w, W2T, b2)


def kernel(x, edge_index, edge_attr, W1, b1, gamma, beta, prelu_w, W2, b2):
    N, H = x.shape
    psum, cnt = _scatter_sum_sc(edge_index.reshape(-1), edge_attr, N)
    return _mlp_tc(
        x, psum, cnt,
        W1.T, b1.reshape(1, -1), gamma.reshape(1, -1), beta.reshape(1, -1),
        prelu_w.reshape(1, 1), W2.T, b2.reshape(1, -1),
    )
